# Initial kernel scaffold; baseline (speedup 1.0000x reference)
import jax
import jax.numpy as jnp
from jax.experimental import pallas as pl


def kernel(label_image, intensity_image):
    raise NotImplementedError("write your pallas kernel here")


# speedup vs baseline: 214.8842x; 214.8842x over previous
import functools

import jax
import jax.numpy as jnp
from jax import lax
from jax.experimental import pallas as pl
from jax.experimental.pallas import tpu as pltpu
from jax.experimental.pallas import tpu_sc as plsc

NLAB = 512
MINI = 0.2
MAXI = 0.8
NC, NS, L = 2, 16, 16
NW = NC * NS
N_TOTAL = 32 * 512 * 512
PER_W = N_TOTAL // NW
CHUNK = 8192
NCHUNK = PER_W // CHUNK
VPC = CHUNK // L

_mesh = plsc.VectorSubcoreMesh(
    core_axis_name="c", subcore_axis_name="s", num_cores=NC, num_subcores=NS
)
_params = pltpu.CompilerParams(needs_layout_passes=False)


def _wid():
    return lax.axis_index("s") * NC + lax.axis_index("c")


@functools.partial(
    pl.kernel,
    out_type=jax.ShapeDtypeStruct((NW, 2 * NLAB), jnp.float32),
    mesh=_mesh,
    compiler_params=_params,
    scratch_types=[
        pltpu.VMEM((CHUNK,), jnp.int32),
        pltpu.VMEM((CHUNK,), jnp.float32),
        pltpu.VMEM((L * NLAB,), jnp.float32),
        pltpu.VMEM((L * NLAB,), jnp.float32),
        pltpu.VMEM((2 * NLAB,), jnp.float32),
    ],
)
def _pass1(lab_hbm, int_hbm, tbl_hbm, lab_v, int_v, sums_v, cnts_v, row_v):
    wid = _wid()
    base = wid * PER_W
    zero16 = jnp.zeros((L,), jnp.float32)
    ones16 = jnp.ones((L,), jnp.float32)
    laneoff = lax.iota(jnp.int32, L) * NLAB

    def zbody(i, _):
        sums_v[pl.ds(i * L, L)] = zero16
        cnts_v[pl.ds(i * L, L)] = zero16
        return 0

    lax.fori_loop(0, (L * NLAB) // L, zbody, 0)

    def chunk_body(c, _):
        off = base + c * CHUNK
        pltpu.sync_copy(lab_hbm.at[pl.ds(off, CHUNK)], lab_v)
        pltpu.sync_copy(int_hbm.at[pl.ds(off, CHUNK)], int_v)

        def body(i, _):
            lab = lab_v[pl.ds(i * L, L)]
            vals = int_v[pl.ds(i * L, L)]
            idx = lab + laneoff
            plsc.addupdate_scatter(sums_v, [idx], vals)
            plsc.addupdate_scatter(cnts_v, [idx], ones16)
            return 0

        lax.fori_loop(0, VPC, body, 0)
        return 0

    lax.fori_loop(0, NCHUNK, chunk_body, 0)

    def red_body(j, _):
        def accs(l, a):
            return a + sums_v[pl.ds(l * NLAB + j * L, L)]

        def accc(l, a):
            return a + cnts_v[pl.ds(l * NLAB + j * L, L)]

        row_v[pl.ds(j * L, L)] = lax.fori_loop(0, L, accs, zero16)
        row_v[pl.ds(NLAB + j * L, L)] = lax.fori_loop(0, L, accc, zero16)
        return 0

    lax.fori_loop(0, NLAB // L, red_body, 0)
    pltpu.sync_copy(row_v, tbl_hbm.at[wid])


@functools.partial(
    pl.kernel,
    out_type=jax.ShapeDtypeStruct((N_TOTAL,), jnp.int32),
    mesh=_mesh,
    compiler_params=_params,
    scratch_types=[
        pltpu.VMEM((NW, 2 * NLAB), jnp.float32),
        pltpu.VMEM((NLAB,), jnp.int32),
        pltpu.VMEM((CHUNK,), jnp.int32),
        pltpu.VMEM((CHUNK,), jnp.int32),
    ],
)
def _pass2(lab_hbm, tbl_hbm, out_hbm, tbl_v, remap_v, lab_v, out_v):
    wid = _wid()
    base = wid * PER_W
    zero16 = jnp.zeros((L,), jnp.float32)
    iota16 = lax.iota(jnp.int32, L)

    pltpu.sync_copy(tbl_hbm, tbl_v)

    def rbody(j, _):
        def accs(w, a):
            return a + tbl_v[w, pl.ds(j * L, L)]

        def accc(w, a):
            return a + tbl_v[w, pl.ds(NLAB + j * L, L)]

        s = lax.fori_loop(0, NW, accs, zero16)
        c = lax.fori_loop(0, NW, accc, zero16)
        mean = s / jnp.maximum(c, 1.0)
        ids = iota16 + j * L
        bad = ((mean < MINI) | (mean > MAXI)) & (ids != 0) & (c > 0.0)
        remap_v[pl.ds(j * L, L)] = jnp.where(bad, 0, ids)
        return 0

    lax.fori_loop(0, NLAB // L, rbody, 0)

    def chunk_body(c, _):
        off = base + c * CHUNK
        pltpu.sync_copy(lab_hbm.at[pl.ds(off, CHUNK)], lab_v)

        def body(i, _):
            lab = lab_v[pl.ds(i * L, L)]
            out_v[pl.ds(i * L, L)] = plsc.load_gather(remap_v, [lab])
            return 0

        lax.fori_loop(0, VPC, body, 0)
        pltpu.sync_copy(out_v, out_hbm.at[pl.ds(off, CHUNK)])
        return 0

    lax.fori_loop(0, NCHUNK, chunk_body, 0)


def kernel(label_image, intensity_image):
    lab = label_image.reshape(-1)
    inten = intensity_image.reshape(-1)
    tbl = _pass1(lab, inten)
    out = _pass2(lab, tbl)
    return out.reshape(label_image.shape)


# speedup vs baseline: 273.2931x; 1.2718x over previous
import functools

import jax
import jax.numpy as jnp
from jax import lax
from jax.experimental import pallas as pl
from jax.experimental.pallas import tpu as pltpu
from jax.experimental.pallas import tpu_sc as plsc

NLAB = 512
MINI = 0.2
MAXI = 0.8
NC, NS, L = 2, 16, 16
NW = NC * NS
N_TOTAL = 32 * 512 * 512
PER_W = N_TOTAL // NW
CHUNK = 16384
NCHUNK = PER_W // CHUNK
VPC = CHUNK // L
U = 8

_mesh = plsc.VectorSubcoreMesh(
    core_axis_name="c", subcore_axis_name="s", num_cores=NC, num_subcores=NS
)
_params = pltpu.CompilerParams(needs_layout_passes=False)


def _wid():
    return lax.axis_index("s") * NC + lax.axis_index("c")


@functools.partial(
    pl.kernel,
    out_type=jax.ShapeDtypeStruct((NW, 2 * NLAB), jnp.float32),
    mesh=_mesh,
    compiler_params=_params,
    scratch_types=[
        pltpu.VMEM((2, CHUNK), jnp.int32),
        pltpu.VMEM((2, CHUNK), jnp.float32),
        pltpu.VMEM((L * NLAB,), jnp.float32),
        pltpu.VMEM((L * NLAB,), jnp.float32),
        pltpu.VMEM((2 * NLAB,), jnp.float32),
        pltpu.SemaphoreType.DMA,
        pltpu.SemaphoreType.DMA,
        pltpu.SemaphoreType.DMA,
        pltpu.SemaphoreType.DMA,
    ],
)
def _pass1(lab_hbm, int_hbm, tbl_hbm, lab_v, int_v, sums_v, cnts_v, row_v,
           sl0, sl1, si0, si1):
    wid = _wid()
    base = wid * PER_W
    sem_lab = (sl0, sl1)
    sem_int = (si0, si1)
    zero16 = jnp.zeros((L,), jnp.float32)
    ones16 = jnp.ones((L,), jnp.float32)
    laneoff = lax.iota(jnp.int32, L) * NLAB

    def zbody(i, _):
        sums_v[pl.ds(i * L, L)] = zero16
        cnts_v[pl.ds(i * L, L)] = zero16
        return 0

    lax.fori_loop(0, (L * NLAB) // L, zbody, 0)

    for b in range(2):
        off = base + b * CHUNK
        pltpu.async_copy(lab_hbm.at[pl.ds(off, CHUNK)], lab_v.at[b], sem_lab[b])
        pltpu.async_copy(int_hbm.at[pl.ds(off, CHUNK)], int_v.at[b], sem_int[b])

    def pair_body(c2, _):
        for b in range(2):
            c = c2 * 2 + b
            off = base + c * CHUNK
            pltpu.make_async_copy(
                lab_hbm.at[pl.ds(off, CHUNK)], lab_v.at[b], sem_lab[b]
            ).wait()
            pltpu.make_async_copy(
                int_hbm.at[pl.ds(off, CHUNK)], int_v.at[b], sem_int[b]
            ).wait()

            def body(i, _):
                o0 = i * (U * L)
                for u in range(U):
                    o = o0 + u * L
                    lab = lab_v[b, pl.ds(o, L)]
                    vals = int_v[b, pl.ds(o, L)]
                    idx = lab + laneoff
                    plsc.addupdate_scatter(sums_v, [idx], vals)
                    plsc.addupdate_scatter(cnts_v, [idx], ones16)
                return 0

            lax.fori_loop(0, VPC // U, body, 0)

            @pl.when(c + 2 < NCHUNK)
            def _():
                off2 = base + (c + 2) * CHUNK
                pltpu.async_copy(
                    lab_hbm.at[pl.ds(off2, CHUNK)], lab_v.at[b], sem_lab[b]
                )
                pltpu.async_copy(
                    int_hbm.at[pl.ds(off2, CHUNK)], int_v.at[b], sem_int[b]
                )
        return 0

    lax.fori_loop(0, NCHUNK // 2, pair_body, 0)

    def red_body(j, _):
        jl = j * L
        s = zero16
        c = zero16
        for l in range(L):
            s = s + sums_v[pl.ds(l * NLAB + jl, L)]
            c = c + cnts_v[pl.ds(l * NLAB + jl, L)]
        row_v[pl.ds(jl, L)] = s
        row_v[pl.ds(NLAB + jl, L)] = c
        return 0

    lax.fori_loop(0, NLAB // L, red_body, 0)
    pltpu.sync_copy(row_v, tbl_hbm.at[wid])


@functools.partial(
    pl.kernel,
    out_type=jax.ShapeDtypeStruct((N_TOTAL,), jnp.int32),
    mesh=_mesh,
    compiler_params=_params,
    scratch_types=[
        pltpu.VMEM((NW, 2 * NLAB), jnp.float32),
        pltpu.VMEM((NLAB,), jnp.int32),
        pltpu.VMEM((2, CHUNK), jnp.int32),
        pltpu.VMEM((2, CHUNK), jnp.int32),
        pltpu.SemaphoreType.DMA,
        pltpu.SemaphoreType.DMA,
        pltpu.SemaphoreType.DMA,
        pltpu.SemaphoreType.DMA,
    ],
)
def _pass2(lab_hbm, tbl_hbm, out_hbm, tbl_v, remap_v, lab_v, out_v,
           sl0, sl1, so0, so1):
    wid = _wid()
    base = wid * PER_W
    sem_lab = (sl0, sl1)
    sem_out = (so0, so1)
    zero16 = jnp.zeros((L,), jnp.float32)
    iota16 = lax.iota(jnp.int32, L)

    pltpu.sync_copy(tbl_hbm, tbl_v)

    def rbody(j, _):
        jl = j * L
        s = zero16
        c = zero16
        for w in range(NW):
            s = s + tbl_v[w, pl.ds(jl, L)]
            c = c + tbl_v[w, pl.ds(NLAB + jl, L)]
        mean = s / jnp.maximum(c, 1.0)
        ids = iota16 + jl
        bad = ((mean < MINI) | (mean > MAXI)) & (ids != 0) & (c > 0.0)
        remap_v[pl.ds(jl, L)] = jnp.where(bad, 0, ids)
        return 0

    lax.fori_loop(0, NLAB // L, rbody, 0)

    for b in range(2):
        off = base + b * CHUNK
        pltpu.async_copy(lab_hbm.at[pl.ds(off, CHUNK)], lab_v.at[b], sem_lab[b])

    def pair_body(c2, _):
        for b in range(2):
            c = c2 * 2 + b
            off = base + c * CHUNK
            pltpu.make_async_copy(
                lab_hbm.at[pl.ds(off, CHUNK)], lab_v.at[b], sem_lab[b]
            ).wait()

            @pl.when(c >= 2)
            def _():
                offp = off - 2 * CHUNK
                pltpu.make_async_copy(
                    out_v.at[b], out_hbm.at[pl.ds(offp, CHUNK)], sem_out[b]
                ).wait()

            def body(i, _):
                o0 = i * (U * L)
                for u in range(U):
                    o = o0 + u * L
                    lab = lab_v[b, pl.ds(o, L)]
                    out_v[b, pl.ds(o, L)] = plsc.load_gather(remap_v, [lab])
                return 0

            lax.fori_loop(0, VPC // U, body, 0)
            pltpu.async_copy(
                out_v.at[b], out_hbm.at[pl.ds(off, CHUNK)], sem_out[b]
            )

            @pl.when(c + 2 < NCHUNK)
            def _():
                off2 = base + (c + 2) * CHUNK
                pltpu.async_copy(
                    lab_hbm.at[pl.ds(off2, CHUNK)], lab_v.at[b], sem_lab[b]
                )
        return 0

    lax.fori_loop(0, NCHUNK // 2, pair_body, 0)

    for b in range(2):
        off = base + (NCHUNK - 2 + b) * CHUNK
        pltpu.make_async_copy(
            out_v.at[b], out_hbm.at[pl.ds(off, CHUNK)], sem_out[b]
        ).wait()


def kernel(label_image, intensity_image):
    lab = label_image.reshape(-1)
    inten = intensity_image.reshape(-1)
    tbl = _pass1(lab, inten)
    out = _pass2(lab, tbl)
    return out.reshape(label_image.shape)


# speedup vs baseline: 773.2424x; 2.8294x over previous
import functools

import jax
import jax.numpy as jnp
from jax import lax
from jax.experimental import pallas as pl
from jax.experimental.pallas import tpu as pltpu
from jax.experimental.pallas import tpu_sc as plsc

NLAB = 512
MINI = 0.2
MAXI = 0.8
NC, NS, L = 2, 16, 16
NW = NC * NS
ZDIM, YDIM, XDIM = 32, 512, 512
R = 32
NCHUNK = YDIM // R
U = 8
GROUPS = XDIM // (U * L)

_mesh = plsc.VectorSubcoreMesh(
    core_axis_name="c", subcore_axis_name="s", num_cores=NC, num_subcores=NS
)
_params = pltpu.CompilerParams(
    needs_layout_passes=False, use_tc_tiling_on_sc=True
)


def _wid():
    return lax.axis_index("s") * NC + lax.axis_index("c")


@functools.partial(
    pl.kernel,
    out_type=jax.ShapeDtypeStruct((NW * 2 * NLAB,), jnp.float32),
    mesh=_mesh,
    compiler_params=_params,
    scratch_types=[
        pltpu.VMEM((2, R, XDIM), jnp.int32),
        pltpu.VMEM((2, R, XDIM), jnp.float32),
        pltpu.VMEM((L * NLAB,), jnp.float32),
        pltpu.VMEM((L * NLAB,), jnp.float32),
        pltpu.VMEM((2 * NLAB,), jnp.float32),
        pltpu.SemaphoreType.DMA,
        pltpu.SemaphoreType.DMA,
        pltpu.SemaphoreType.DMA,
        pltpu.SemaphoreType.DMA,
    ],
)
def _pass1(lab_hbm, int_hbm, tbl_hbm, lab_v, int_v, sums_v, cnts_v, row_v,
           sl0, sl1, si0, si1):
    wid = _wid()
    sem_lab = (sl0, sl1)
    sem_int = (si0, si1)
    zero16 = jnp.zeros((L,), jnp.float32)
    ones16 = jnp.ones((L,), jnp.float32)
    laneoff = lax.iota(jnp.int32, L) * NLAB

    def zbody(i, _):
        sums_v[pl.ds(i * L, L)] = zero16
        cnts_v[pl.ds(i * L, L)] = zero16
        return 0

    lax.fori_loop(0, (L * NLAB) // L, zbody, 0)

    for b in range(2):
        r0 = b * R
        pltpu.async_copy(
            lab_hbm.at[wid, pl.ds(r0, R)], lab_v.at[b], sem_lab[b]
        )
        pltpu.async_copy(
            int_hbm.at[wid, pl.ds(r0, R)], int_v.at[b], sem_int[b]
        )

    def pair_body(c2, _):
        for b in range(2):
            c = c2 * 2 + b
            r0 = c * R
            pltpu.make_async_copy(
                lab_hbm.at[wid, pl.ds(r0, R)], lab_v.at[b], sem_lab[b]
            ).wait()
            pltpu.make_async_copy(
                int_hbm.at[wid, pl.ds(r0, R)], int_v.at[b], sem_int[b]
            ).wait()

            def row_body(r, _):
                for g in range(GROUPS):
                    g0 = g * U * L
                    labs = [
                        lab_v[b, r, pl.ds(g0 + u * L, L)] for u in range(U)
                    ]
                    vals = [
                        int_v[b, r, pl.ds(g0 + u * L, L)] for u in range(U)
                    ]
                    idxs = [lab + laneoff for lab in labs]
                    for u in range(U):
                        plsc.addupdate_scatter(sums_v, [idxs[u]], vals[u])
                    for u in range(U):
                        plsc.addupdate_scatter(cnts_v, [idxs[u]], ones16)
                return 0

            lax.fori_loop(0, R, row_body, 0)

            @pl.when(c + 2 < NCHUNK)
            def _():
                r2 = (c + 2) * R
                pltpu.async_copy(
                    lab_hbm.at[wid, pl.ds(r2, R)], lab_v.at[b], sem_lab[b]
                )
                pltpu.async_copy(
                    int_hbm.at[wid, pl.ds(r2, R)], int_v.at[b], sem_int[b]
                )
        return 0

    lax.fori_loop(0, NCHUNK // 2, pair_body, 0)

    def red_body(j, _):
        jl = j * L
        s = zero16
        c = zero16
        for l in range(L):
            s = s + sums_v[pl.ds(l * NLAB + jl, L)]
            c = c + cnts_v[pl.ds(l * NLAB + jl, L)]
        row_v[pl.ds(jl, L)] = s
        row_v[pl.ds(NLAB + jl, L)] = c
        return 0

    lax.fori_loop(0, NLAB // L, red_body, 0)
    pltpu.sync_copy(row_v, tbl_hbm.at[pl.ds(wid * 2 * NLAB, 2 * NLAB)])


@functools.partial(
    pl.kernel,
    out_type=jax.ShapeDtypeStruct((ZDIM, YDIM, XDIM), jnp.int32),
    mesh=_mesh,
    compiler_params=_params,
    scratch_types=[
        pltpu.VMEM((NW * 2 * NLAB,), jnp.float32),
        pltpu.VMEM((NLAB,), jnp.int32),
        pltpu.VMEM((2, R, XDIM), jnp.int32),
        pltpu.VMEM((2, R, XDIM), jnp.int32),
        pltpu.SemaphoreType.DMA,
        pltpu.SemaphoreType.DMA,
        pltpu.SemaphoreType.DMA,
        pltpu.SemaphoreType.DMA,
    ],
)
def _pass2(lab_hbm, tbl_hbm, out_hbm, tbl_v, remap_v, lab_v, out_v,
           sl0, sl1, so0, so1):
    wid = _wid()
    sem_lab = (sl0, sl1)
    sem_out = (so0, so1)
    zero16 = jnp.zeros((L,), jnp.float32)
    iota16 = lax.iota(jnp.int32, L)

    pltpu.sync_copy(tbl_hbm, tbl_v)

    def rbody(j, _):
        jl = j * L
        s = zero16
        c = zero16
        for w in range(NW):
            s = s + tbl_v[pl.ds(w * 2 * NLAB + jl, L)]
            c = c + tbl_v[pl.ds(w * 2 * NLAB + NLAB + jl, L)]
        mean = s / jnp.maximum(c, 1.0)
        ids = iota16 + jl
        bad = ((mean < MINI) | (mean > MAXI)) & (ids != 0) & (c > 0.0)
        remap_v[pl.ds(jl, L)] = jnp.where(bad, 0, ids)
        return 0

    lax.fori_loop(0, NLAB // L, rbody, 0)

    for b in range(2):
        r0 = b * R
        pltpu.async_copy(
            lab_hbm.at[wid, pl.ds(r0, R)], lab_v.at[b], sem_lab[b]
        )

    def pair_body(c2, _):
        for b in range(2):
            c = c2 * 2 + b
            r0 = c * R
            pltpu.make_async_copy(
                lab_hbm.at[wid, pl.ds(r0, R)], lab_v.at[b], sem_lab[b]
            ).wait()

            @pl.when(c >= 2)
            def _():
                rp = r0 - 2 * R
                pltpu.make_async_copy(
                    out_v.at[b], out_hbm.at[wid, pl.ds(rp, R)], sem_out[b]
                ).wait()

            def row_body(r, _):
                for g in range(GROUPS):
                    g0 = g * U * L
                    labs = [
                        lab_v[b, r, pl.ds(g0 + u * L, L)] for u in range(U)
                    ]
                    news = [plsc.load_gather(remap_v, [lab]) for lab in labs]
                    for u in range(U):
                        out_v[b, r, pl.ds(g0 + u * L, L)] = news[u]
                return 0

            lax.fori_loop(0, R, row_body, 0)
            pltpu.async_copy(
                out_v.at[b], out_hbm.at[wid, pl.ds(r0, R)], sem_out[b]
            )

            @pl.when(c + 2 < NCHUNK)
            def _():
                r2 = (c + 2) * R
                pltpu.async_copy(
                    lab_hbm.at[wid, pl.ds(r2, R)], lab_v.at[b], sem_lab[b]
                )
        return 0

    lax.fori_loop(0, NCHUNK // 2, pair_body, 0)

    for b in range(2):
        r0 = (NCHUNK - 2 + b) * R
        pltpu.make_async_copy(
            out_v.at[b], out_hbm.at[wid, pl.ds(r0, R)], sem_out[b]
        ).wait()


def kernel(label_image, intensity_image):
    tbl = _pass1(label_image, intensity_image)
    return _pass2(label_image, tbl)


# speedup vs baseline: 791.8427x; 1.0241x over previous
import functools

import jax
import jax.numpy as jnp
from jax import lax
from jax.experimental import pallas as pl
from jax.experimental.pallas import tpu as pltpu
from jax.experimental.pallas import tpu_sc as plsc

NLAB = 512
MINI = 0.2
MAXI = 0.8
NC, NS, L = 2, 16, 16
NW = NC * NS
ZDIM, YDIM, XDIM = 32, 512, 512
R = 32
NCHUNK = YDIM // R
U = 8
GROUPS = XDIM // (U * L)

_mesh = plsc.VectorSubcoreMesh(
    core_axis_name="c", subcore_axis_name="s", num_cores=NC, num_subcores=NS
)
_params = pltpu.CompilerParams(
    needs_layout_passes=False, use_tc_tiling_on_sc=True
)


def _wid():
    return lax.axis_index("s") * NC + lax.axis_index("c")


@functools.partial(
    pl.kernel,
    out_type=jax.ShapeDtypeStruct((NW * 2 * NLAB,), jnp.float32),
    mesh=_mesh,
    compiler_params=_params,
    scratch_types=[
        pltpu.VMEM((2, R, XDIM), jnp.int32),
        pltpu.VMEM((2, R, XDIM), jnp.float32),
        pltpu.VMEM((L * NLAB,), jnp.float32),
        pltpu.VMEM((L * NLAB,), jnp.float32),
        pltpu.VMEM((2 * NLAB,), jnp.float32),
        pltpu.SemaphoreType.DMA,
        pltpu.SemaphoreType.DMA,
        pltpu.SemaphoreType.DMA,
        pltpu.SemaphoreType.DMA,
    ],
)
def _pass1(lab_hbm, int_hbm, tbl_hbm, lab_v, int_v, sums_v, cnts_v, row_v,
           sl0, sl1, si0, si1):
    wid = _wid()
    sem_lab = (sl0, sl1)
    sem_int = (si0, si1)
    zero16 = jnp.zeros((L,), jnp.float32)
    ones16 = jnp.ones((L,), jnp.float32)
    laneoff = lax.iota(jnp.int32, L) * NLAB

    def zbody(i, _):
        sums_v[pl.ds(i * L, L)] = zero16
        cnts_v[pl.ds(i * L, L)] = zero16
        return 0

    lax.fori_loop(0, (L * NLAB) // L, zbody, 0)

    for b in range(2):
        r0 = b * R
        pltpu.async_copy(
            lab_hbm.at[wid, pl.ds(r0, R)], lab_v.at[b], sem_lab[b]
        )
        pltpu.async_copy(
            int_hbm.at[wid, pl.ds(r0, R)], int_v.at[b], sem_int[b]
        )

    def pair_body(c2, _):
        for b in range(2):
            c = c2 * 2 + b
            r0 = c * R
            pltpu.make_async_copy(
                lab_hbm.at[wid, pl.ds(r0, R)], lab_v.at[b], sem_lab[b]
            ).wait()
            pltpu.make_async_copy(
                int_hbm.at[wid, pl.ds(r0, R)], int_v.at[b], sem_int[b]
            ).wait()

            @plsc.parallel_loop(0, R)
            def row_body(r):
                for g in range(GROUPS):
                    g0 = g * U * L
                    labs = [
                        lab_v[b, r, pl.ds(g0 + u * L, L)] for u in range(U)
                    ]
                    vals = [
                        int_v[b, r, pl.ds(g0 + u * L, L)] for u in range(U)
                    ]
                    idxs = [lab + laneoff for lab in labs]
                    for u in range(U):
                        plsc.addupdate_scatter(sums_v, [idxs[u]], vals[u])
                    for u in range(U):
                        plsc.addupdate_scatter(cnts_v, [idxs[u]], ones16)

            @pl.when(c + 2 < NCHUNK)
            def _():
                r2 = (c + 2) * R
                pltpu.async_copy(
                    lab_hbm.at[wid, pl.ds(r2, R)], lab_v.at[b], sem_lab[b]
                )
                pltpu.async_copy(
                    int_hbm.at[wid, pl.ds(r2, R)], int_v.at[b], sem_int[b]
                )
        return 0

    lax.fori_loop(0, NCHUNK // 2, pair_body, 0)

    def red_body(j, _):
        jl = j * L
        s = zero16
        c = zero16
        for l in range(L):
            s = s + sums_v[pl.ds(l * NLAB + jl, L)]
            c = c + cnts_v[pl.ds(l * NLAB + jl, L)]
        row_v[pl.ds(jl, L)] = s
        row_v[pl.ds(NLAB + jl, L)] = c
        return 0

    lax.fori_loop(0, NLAB // L, red_body, 0)
    pltpu.sync_copy(row_v, tbl_hbm.at[pl.ds(wid * 2 * NLAB, 2 * NLAB)])


@functools.partial(
    pl.kernel,
    out_type=jax.ShapeDtypeStruct((ZDIM, YDIM, XDIM), jnp.int32),
    mesh=_mesh,
    compiler_params=_params,
    scratch_types=[
        pltpu.VMEM((NW * 2 * NLAB,), jnp.float32),
        pltpu.VMEM((NLAB,), jnp.int32),
        pltpu.VMEM((2, R, XDIM), jnp.int32),
        pltpu.VMEM((2, R, XDIM), jnp.int32),
        pltpu.SemaphoreType.DMA,
        pltpu.SemaphoreType.DMA,
        pltpu.SemaphoreType.DMA,
        pltpu.SemaphoreType.DMA,
    ],
)
def _pass2(lab_hbm, tbl_hbm, out_hbm, tbl_v, remap_v, lab_v, out_v,
           sl0, sl1, so0, so1):
    wid = _wid()
    sem_lab = (sl0, sl1)
    sem_out = (so0, so1)
    zero16 = jnp.zeros((L,), jnp.float32)
    iota16 = lax.iota(jnp.int32, L)

    pltpu.sync_copy(tbl_hbm, tbl_v)

    def rbody(j, _):
        jl = j * L
        s = zero16
        c = zero16
        for w in range(NW):
            s = s + tbl_v[pl.ds(w * 2 * NLAB + jl, L)]
            c = c + tbl_v[pl.ds(w * 2 * NLAB + NLAB + jl, L)]
        mean = s / jnp.maximum(c, 1.0)
        ids = iota16 + jl
        bad = ((mean < MINI) | (mean > MAXI)) & (ids != 0) & (c > 0.0)
        remap_v[pl.ds(jl, L)] = jnp.where(bad, 0, ids)
        return 0

    lax.fori_loop(0, NLAB // L, rbody, 0)

    for b in range(2):
        r0 = b * R
        pltpu.async_copy(
            lab_hbm.at[wid, pl.ds(r0, R)], lab_v.at[b], sem_lab[b]
        )

    def pair_body(c2, _):
        for b in range(2):
            c = c2 * 2 + b
            r0 = c * R
            pltpu.make_async_copy(
                lab_hbm.at[wid, pl.ds(r0, R)], lab_v.at[b], sem_lab[b]
            ).wait()

            @pl.when(c >= 2)
            def _():
                rp = r0 - 2 * R
                pltpu.make_async_copy(
                    out_v.at[b], out_hbm.at[wid, pl.ds(rp, R)], sem_out[b]
                ).wait()

            @plsc.parallel_loop(0, R)
            def row_body(r):
                for g in range(GROUPS):
                    g0 = g * U * L
                    labs = [
                        lab_v[b, r, pl.ds(g0 + u * L, L)] for u in range(U)
                    ]
                    news = [plsc.load_gather(remap_v, [lab]) for lab in labs]
                    for u in range(U):
                        out_v[b, r, pl.ds(g0 + u * L, L)] = news[u]
            pltpu.async_copy(
                out_v.at[b], out_hbm.at[wid, pl.ds(r0, R)], sem_out[b]
            )

            @pl.when(c + 2 < NCHUNK)
            def _():
                r2 = (c + 2) * R
                pltpu.async_copy(
                    lab_hbm.at[wid, pl.ds(r2, R)], lab_v.at[b], sem_lab[b]
                )
        return 0

    lax.fori_loop(0, NCHUNK // 2, pair_body, 0)

    for b in range(2):
        r0 = (NCHUNK - 2 + b) * R
        pltpu.make_async_copy(
            out_v.at[b], out_hbm.at[wid, pl.ds(r0, R)], sem_out[b]
        ).wait()


def kernel(label_image, intensity_image):
    tbl = _pass1(label_image, intensity_image)
    return _pass2(label_image, tbl)


# speedup vs baseline: 810.5742x; 1.0237x over previous
import functools

import jax
import jax.numpy as jnp
from jax import lax
from jax.experimental import pallas as pl
from jax.experimental.pallas import tpu as pltpu
from jax.experimental.pallas import tpu_sc as plsc

NLAB = 512
MINI = 0.2
MAXI = 0.8
NC, NS, L = 2, 16, 16
NW = NC * NS
ZDIM, YDIM, XDIM = 32, 512, 512
R = 32
NCHUNK = YDIM // R
U = 8
GROUPS = XDIM // (U * L)

_mesh = plsc.VectorSubcoreMesh(
    core_axis_name="c", subcore_axis_name="s", num_cores=NC, num_subcores=NS
)
_params = pltpu.CompilerParams(
    needs_layout_passes=False, use_tc_tiling_on_sc=True
)


def _wid():
    return lax.axis_index("s") * NC + lax.axis_index("c")


@functools.partial(
    pl.kernel,
    out_type=jax.ShapeDtypeStruct((NW * 2 * NLAB,), jnp.float32),
    mesh=_mesh,
    compiler_params=_params,
    scratch_types=[
        pltpu.VMEM((2, R, XDIM), jnp.int32),
        pltpu.VMEM((2, R, XDIM), jnp.float32),
        pltpu.VMEM((NLAB,), jnp.float32),
        pltpu.VMEM((NLAB,), jnp.float32),
        pltpu.VMEM((2 * NLAB,), jnp.float32),
        pltpu.SemaphoreType.DMA,
        pltpu.SemaphoreType.DMA,
        pltpu.SemaphoreType.DMA,
        pltpu.SemaphoreType.DMA,
    ],
)
def _pass1(lab_hbm, int_hbm, tbl_hbm, lab_v, int_v, sums_v, cnts_v, row_v,
           sl0, sl1, si0, si1):
    wid = _wid()
    sem_lab = (sl0, sl1)
    sem_int = (si0, si1)
    zero16 = jnp.zeros((L,), jnp.float32)
    ones16 = jnp.ones((L,), jnp.float32)

    def zbody(i, _):
        sums_v[pl.ds(i * L, L)] = zero16
        cnts_v[pl.ds(i * L, L)] = zero16
        return 0

    lax.fori_loop(0, NLAB // L, zbody, 0)

    for b in range(2):
        r0 = b * R
        pltpu.async_copy(
            lab_hbm.at[wid, pl.ds(r0, R)], lab_v.at[b], sem_lab[b]
        )
        pltpu.async_copy(
            int_hbm.at[wid, pl.ds(r0, R)], int_v.at[b], sem_int[b]
        )

    def pair_body(c2, _):
        for b in range(2):
            c = c2 * 2 + b
            r0 = c * R
            pltpu.make_async_copy(
                lab_hbm.at[wid, pl.ds(r0, R)], lab_v.at[b], sem_lab[b]
            ).wait()
            pltpu.make_async_copy(
                int_hbm.at[wid, pl.ds(r0, R)], int_v.at[b], sem_int[b]
            ).wait()

            @plsc.parallel_loop(0, R)
            def row_body(r):
                for g in range(GROUPS):
                    g0 = g * U * L
                    labs = [
                        lab_v[b, r, pl.ds(g0 + u * L, L)] for u in range(U)
                    ]
                    vals = [
                        int_v[b, r, pl.ds(g0 + u * L, L)] for u in range(U)
                    ]
                    idxs = labs
                    for u in range(U):
                        plsc.addupdate_scatter(sums_v, [idxs[u]], vals[u])
                    for u in range(U):
                        plsc.addupdate_scatter(cnts_v, [idxs[u]], ones16)

            @pl.when(c + 2 < NCHUNK)
            def _():
                r2 = (c + 2) * R
                pltpu.async_copy(
                    lab_hbm.at[wid, pl.ds(r2, R)], lab_v.at[b], sem_lab[b]
                )
                pltpu.async_copy(
                    int_hbm.at[wid, pl.ds(r2, R)], int_v.at[b], sem_int[b]
                )
        return 0

    lax.fori_loop(0, NCHUNK // 2, pair_body, 0)

    def red_body(j, _):
        jl = j * L
        row_v[pl.ds(jl, L)] = sums_v[pl.ds(jl, L)]
        row_v[pl.ds(NLAB + jl, L)] = cnts_v[pl.ds(jl, L)]
        return 0

    lax.fori_loop(0, NLAB // L, red_body, 0)
    pltpu.sync_copy(row_v, tbl_hbm.at[pl.ds(wid * 2 * NLAB, 2 * NLAB)])


@functools.partial(
    pl.kernel,
    out_type=jax.ShapeDtypeStruct((ZDIM, YDIM, XDIM), jnp.int32),
    mesh=_mesh,
    compiler_params=_params,
    scratch_types=[
        pltpu.VMEM((NW * 2 * NLAB,), jnp.float32),
        pltpu.VMEM((NLAB,), jnp.int32),
        pltpu.VMEM((2, R, XDIM), jnp.int32),
        pltpu.VMEM((2, R, XDIM), jnp.int32),
        pltpu.SemaphoreType.DMA,
        pltpu.SemaphoreType.DMA,
        pltpu.SemaphoreType.DMA,
        pltpu.SemaphoreType.DMA,
    ],
)
def _pass2(lab_hbm, tbl_hbm, out_hbm, tbl_v, remap_v, lab_v, out_v,
           sl0, sl1, so0, so1):
    wid = _wid()
    sem_lab = (sl0, sl1)
    sem_out = (so0, so1)
    zero16 = jnp.zeros((L,), jnp.float32)
    iota16 = lax.iota(jnp.int32, L)

    pltpu.sync_copy(tbl_hbm, tbl_v)

    def rbody(j, _):
        jl = j * L
        s = zero16
        c = zero16
        for w in range(NW):
            s = s + tbl_v[pl.ds(w * 2 * NLAB + jl, L)]
            c = c + tbl_v[pl.ds(w * 2 * NLAB + NLAB + jl, L)]
        mean = s / jnp.maximum(c, 1.0)
        ids = iota16 + jl
        bad = ((mean < MINI) | (mean > MAXI)) & (ids != 0) & (c > 0.0)
        remap_v[pl.ds(jl, L)] = jnp.where(bad, 0, ids)
        return 0

    lax.fori_loop(0, NLAB // L, rbody, 0)

    for b in range(2):
        r0 = b * R
        pltpu.async_copy(
            lab_hbm.at[wid, pl.ds(r0, R)], lab_v.at[b], sem_lab[b]
        )

    def pair_body(c2, _):
        for b in range(2):
            c = c2 * 2 + b
            r0 = c * R
            pltpu.make_async_copy(
                lab_hbm.at[wid, pl.ds(r0, R)], lab_v.at[b], sem_lab[b]
            ).wait()

            @pl.when(c >= 2)
            def _():
                rp = r0 - 2 * R
                pltpu.make_async_copy(
                    out_v.at[b], out_hbm.at[wid, pl.ds(rp, R)], sem_out[b]
                ).wait()

            @plsc.parallel_loop(0, R)
            def row_body(r):
                for g in range(GROUPS):
                    g0 = g * U * L
                    labs = [
                        lab_v[b, r, pl.ds(g0 + u * L, L)] for u in range(U)
                    ]
                    news = [plsc.load_gather(remap_v, [lab]) for lab in labs]
                    for u in range(U):
                        out_v[b, r, pl.ds(g0 + u * L, L)] = news[u]
            pltpu.async_copy(
                out_v.at[b], out_hbm.at[wid, pl.ds(r0, R)], sem_out[b]
            )

            @pl.when(c + 2 < NCHUNK)
            def _():
                r2 = (c + 2) * R
                pltpu.async_copy(
                    lab_hbm.at[wid, pl.ds(r2, R)], lab_v.at[b], sem_lab[b]
                )
        return 0

    lax.fori_loop(0, NCHUNK // 2, pair_body, 0)

    for b in range(2):
        r0 = (NCHUNK - 2 + b) * R
        pltpu.make_async_copy(
            out_v.at[b], out_hbm.at[wid, pl.ds(r0, R)], sem_out[b]
        ).wait()


def kernel(label_image, intensity_image):
    tbl = _pass1(label_image, intensity_image)
    return _pass2(label_image, tbl)
